# Initial kernel scaffold; baseline (speedup 1.0000x reference)
import jax
import jax.numpy as jnp
from jax.experimental import pallas as pl


def kernel(x, edge_index, edge_attr, type_emb, enc_W, enc_b, W1, att_src1, att_dst1, We1, att_e1, b1, W2, att_src2, att_dst2, We2, att_e2, b2, W3, b3, W4, b4):
    raise NotImplementedError("write your pallas kernel here")


# speedup vs baseline: 2.9156x; 2.9156x over previous
import functools
import jax
import jax.numpy as jnp
from jax.experimental import pallas as pl

N = 50000
E = 800000
NUM_TYPES = 16
EMB = 64
HID = 16
HEADS = 4
EDGE_DIM = 3

EBLK = 6400


def _mlp_body(hmid_ref, W4_ref, b4_ref, out_ref):
    out_ref[...] = hmid_ref[...] @ W4_ref[...] + b4_ref[...]


def _edge_mlp_pallas(hmid, W4, b4):
    return pl.pallas_call(
        _mlp_body,
        grid=(E // EBLK,),
        in_specs=[
            pl.BlockSpec((EBLK, HID), lambda i: (i, 0)),
            pl.BlockSpec((HID, 3), lambda i: (0, 0)),
            pl.BlockSpec((1, 3), lambda i: (0, 0)),
        ],
        out_specs=pl.BlockSpec((EBLK, 3), lambda i: (i, 0)),
        out_shape=jax.ShapeDtypeStruct((E, 3), jnp.float32),
    )(hmid, W4, b4.reshape(1, 3))


def kernel(x, edge_index, edge_attr, type_emb, enc_W, enc_b, W1, att_src1, att_dst1, We1, att_e1, b1, W2, att_src2, att_dst2, We2, att_e2, b2, W3, b3, W4, b4):
    src, dst = edge_index[0], edge_index[1]
    t = x[:, 0]

    h0_table = jax.nn.relu(type_emb @ enc_W + enc_b)
    xh1_table = (h0_table @ W1).reshape(NUM_TYPES, HEADS, HID)
    as1_t = (xh1_table * att_src1[None]).sum(-1)
    ad1_t = (xh1_table * att_dst1[None]).sum(-1)
    Ae1 = (We1.reshape(EDGE_DIM, HEADS, HID) * att_e1[None]).sum(-1)
    Ae2 = (We2.reshape(EDGE_DIM, 1, HID) * att_e2[None]).sum(-1)
    ae1 = edge_attr @ Ae1
    ae2 = (edge_attr @ Ae2)[:, 0]

    tsrc = t[src]
    tdst = t[dst]
    alpha1 = jax.nn.leaky_relu(as1_t[tsrc] + ad1_t[tdst] + ae1, negative_slope=0.2)
    ex1 = jnp.exp(alpha1)
    flat_idx = (dst * (HEADS * NUM_TYPES))[:, None] + jnp.arange(HEADS)[None] * NUM_TYPES + tsrc[:, None]
    S = jnp.zeros((N * HEADS * NUM_TYPES,), jnp.float32).at[flat_idx.reshape(-1)].add(ex1.reshape(-1))
    S = S.reshape(N, HEADS, NUM_TYPES)

    denom1 = S.sum(-1)
    out1 = jnp.einsum('nht,thc->nhc', S, xh1_table) / (denom1[..., None] + 1e-16)
    h1 = jax.nn.elu(out1.reshape(N, HEADS * HID) + b1)
    xh2 = h1 @ W2
    as2 = xh2 @ att_src2[0]
    ad2 = xh2 @ att_dst2[0]

    alpha2 = jax.nn.leaky_relu(as2[src] + ad2[dst] + ae2, negative_slope=0.2)
    ex2 = jnp.exp(alpha2)
    denom2 = jnp.zeros((N,), jnp.float32).at[dst].add(ex2)
    numer2 = jnp.zeros((N, HID), jnp.float32).at[dst].add(ex2[:, None] * xh2[src])

    h2 = jax.nn.elu(numer2 / (denom2[:, None] + 1e-16) + b2)
    A = h2 @ W3[:HID]
    B = h2 @ W3[HID:]

    hmid = jax.nn.relu(A[src] + B[dst] + b3)

    return _edge_mlp_pallas(hmid, W4, b4)


# speedup vs baseline: 3.1785x; 1.0902x over previous
import functools
import jax
import jax.numpy as jnp
from jax import lax
from jax.experimental import pallas as pl
from jax.experimental.pallas import tpu as pltpu
from jax.experimental.pallas import tpu_sc as plsc

N = 50000
E = 800000
NUM_TYPES = 16
EMB = 64
HID = 16
HEADS = 4
EDGE_DIM = 3

NC = 2
NS = 16
NW = NC * NS

NP = 50048
EP = 819200
KE = 1600

EBLK = 8192


def _sc_mesh():
    return plsc.VectorSubcoreMesh(
        core_axis_name="c", subcore_axis_name="s", num_cores=NC, num_subcores=NS)


def _p5_body(A_hbm, B_hbm, src_hbm, dst_hbm, b3_hbm, out_hbm,
             src_v, dst_v, rowsA_v, rowsB_v, b3_v, sem1, sem2):
    c = lax.axis_index("c")
    s = lax.axis_index("s")
    wid = c * NS + s
    pltpu.sync_copy(b3_hbm, b3_v)

    def chunk(k, carry):
        base = wid * (EP // NW) + k * KE
        pltpu.sync_copy(src_hbm.at[pl.ds(base, KE)], src_v)
        pltpu.sync_copy(dst_hbm.at[pl.ds(base, KE)], dst_v)
        cpA = pltpu.async_copy(A_hbm.at[src_v], rowsA_v, sem1)
        cpB = pltpu.async_copy(B_hbm.at[dst_v], rowsB_v, sem2)
        cpA.wait()
        cpB.wait()
        b3 = b3_v[...]

        def row(i, carry2):
            rowsA_v[i] = jnp.maximum(rowsA_v[i] + rowsB_v[i] + b3, 0.0)
            return carry2

        lax.fori_loop(0, KE, row, 0)
        pltpu.sync_copy(rowsA_v, out_hbm.at[pl.ds(base, KE)])
        return carry

    lax.fori_loop(0, EP // NW // KE, chunk, 0)


def _p5_edge_gather(A_pad, B_pad, src_pad, dst_pad, b3):
    f = pl.kernel(
        _p5_body,
        out_type=jax.ShapeDtypeStruct((EP, HID), jnp.float32),
        mesh=_sc_mesh(),
        compiler_params=pltpu.CompilerParams(use_tc_tiling_on_sc=False),
        scratch_types=[
            pltpu.VMEM((KE,), jnp.int32),
            pltpu.VMEM((KE,), jnp.int32),
            pltpu.VMEM((KE, HID), jnp.float32),
            pltpu.VMEM((KE, HID), jnp.float32),
            pltpu.VMEM((HID,), jnp.float32),
            pltpu.SemaphoreType.DMA,
            pltpu.SemaphoreType.DMA,
        ],
    )
    return f(A_pad, B_pad, src_pad, dst_pad, b3)


def _mlp_body(hmid_ref, W4_ref, b4_ref, out_ref):
    out_ref[...] = hmid_ref[...] @ W4_ref[...] + b4_ref[...]


def _edge_mlp_pallas(hmid, W4, b4):
    return pl.pallas_call(
        _mlp_body,
        grid=(EP // EBLK,),
        in_specs=[
            pl.BlockSpec((EBLK, HID), lambda i: (i, 0)),
            pl.BlockSpec((HID, 3), lambda i: (0, 0)),
            pl.BlockSpec((1, 3), lambda i: (0, 0)),
        ],
        out_specs=pl.BlockSpec((EBLK, 3), lambda i: (i, 0)),
        out_shape=jax.ShapeDtypeStruct((EP, 3), jnp.float32),
    )(hmid, W4, b4.reshape(1, 3))


def kernel(x, edge_index, edge_attr, type_emb, enc_W, enc_b, W1, att_src1, att_dst1, We1, att_e1, b1, W2, att_src2, att_dst2, We2, att_e2, b2, W3, b3, W4, b4):
    src, dst = edge_index[0], edge_index[1]
    t = x[:, 0]

    h0_table = jax.nn.relu(type_emb @ enc_W + enc_b)
    xh1_table = (h0_table @ W1).reshape(NUM_TYPES, HEADS, HID)
    as1_t = (xh1_table * att_src1[None]).sum(-1)
    ad1_t = (xh1_table * att_dst1[None]).sum(-1)
    Ae1 = (We1.reshape(EDGE_DIM, HEADS, HID) * att_e1[None]).sum(-1)
    Ae2 = (We2.reshape(EDGE_DIM, 1, HID) * att_e2[None]).sum(-1)
    ae1 = edge_attr @ Ae1
    ae2 = (edge_attr @ Ae2)[:, 0]

    tsrc = t[src]
    tdst = t[dst]
    alpha1 = jax.nn.leaky_relu(as1_t[tsrc] + ad1_t[tdst] + ae1, negative_slope=0.2)
    ex1 = jnp.exp(alpha1)
    flat_idx = (dst * (HEADS * NUM_TYPES))[:, None] + jnp.arange(HEADS)[None] * NUM_TYPES + tsrc[:, None]
    S = jnp.zeros((N * HEADS * NUM_TYPES,), jnp.float32).at[flat_idx.reshape(-1)].add(ex1.reshape(-1))
    S = S.reshape(N, HEADS, NUM_TYPES)

    denom1 = S.sum(-1)
    out1 = jnp.einsum('nht,thc->nhc', S, xh1_table) / (denom1[..., None] + 1e-16)
    h1 = jax.nn.elu(out1.reshape(N, HEADS * HID) + b1)
    xh2 = h1 @ W2
    as2 = xh2 @ att_src2[0]
    ad2 = xh2 @ att_dst2[0]

    alpha2 = jax.nn.leaky_relu(as2[src] + ad2[dst] + ae2, negative_slope=0.2)
    ex2 = jnp.exp(alpha2)
    denom2 = jnp.zeros((N,), jnp.float32).at[dst].add(ex2)
    numer2 = jnp.zeros((N, HID), jnp.float32).at[dst].add(ex2[:, None] * xh2[src])

    h2 = jax.nn.elu(numer2 / (denom2[:, None] + 1e-16) + b2)
    A = h2 @ W3[:HID]
    B = h2 @ W3[HID:]

    A_pad = jnp.concatenate([A, jnp.zeros((NP - N, HID), jnp.float32)])
    B_pad = jnp.concatenate([B, jnp.zeros((NP - N, HID), jnp.float32)])
    src_pad = jnp.concatenate([src, jnp.full((EP - E,), N, src.dtype)])
    dst_pad = jnp.concatenate([dst, jnp.full((EP - E,), N, dst.dtype)])
    hmid = _p5_edge_gather(A_pad, B_pad, src_pad, dst_pad, b3)

    return _edge_mlp_pallas(hmid, W4, b4)[:E]


# speedup vs baseline: 19.8801x; 6.2546x over previous
import functools
import jax
import jax.numpy as jnp
from jax import lax
from jax.experimental import pallas as pl
from jax.experimental.pallas import tpu as pltpu
from jax.experimental.pallas import tpu_sc as plsc

N = 50000
E = 800000
NUM_TYPES = 16
EMB = 64
HID = 16
HEADS = 4
EDGE_DIM = 3

NC = 2
NS = 16
NW = NC * NS

NP = 50048
EP = 819200
KE = 1600

EBLK = 8192


def _sc_mesh():
    return plsc.VectorSubcoreMesh(
        core_axis_name="c", subcore_axis_name="s", num_cores=NC, num_subcores=NS)


def _splat_i32(v):
    return jnp.zeros((16,), jnp.int32) + v


def _iota16():
    return lax.iota(jnp.int32, 16)


KP1 = 800
P1_CHUNKS = EP // NS // KP1
P1_TILE_WORDS = NP * 32 // NS
P1_ZERO = 6256


def _p1_body(x_hbm, src_hbm, dst_hbm, ae_hbm, astab_hbm, adtab_hbm, S_out,
             src_v, dst_v, tsrc_v, tdst_v, ae_v, astab_v, adtab_v,
             val_v, idx_v, zero_v, S_sp, sem1, sem2):
    c = lax.axis_index("c")
    s = lax.axis_index("s")

    def zfill(i, carry):
        zero_v[pl.ds(i * 16, 16)] = jnp.zeros((16,), jnp.float32)
        return carry

    lax.fori_loop(0, P1_ZERO // 16, zfill, 0)

    def zcopy(k, carry):
        pltpu.sync_copy(zero_v, S_sp.at[pl.ds(s * P1_TILE_WORDS + k * P1_ZERO, P1_ZERO)])
        return carry

    lax.fori_loop(0, NS, zcopy, 0)
    pltpu.sync_copy(astab_hbm, astab_v)
    pltpu.sync_copy(adtab_hbm, adtab_v)
    plsc.subcore_barrier()

    def chunk(k, carry):
        base = s * (EP // NS) + k * KP1
        pltpu.sync_copy(src_hbm.at[pl.ds(base, KP1)], src_v)
        pltpu.sync_copy(dst_hbm.at[pl.ds(base, KP1)], dst_v)
        cp1 = pltpu.async_copy(x_hbm.at[src_v], tsrc_v, sem1)
        cp2 = pltpu.async_copy(x_hbm.at[dst_v], tdst_v, sem2)
        pltpu.sync_copy(ae_hbm.at[pl.ds(base, KP1)], ae_v)
        cp1.wait()
        cp2.wait()

        def group(g, carry2):
            gsl = pl.ds(g * 16, 16)
            tsrc16 = tsrc_v[gsl]
            tdst16 = tdst_v[gsl]
            dst16 = dst_v[gsl]
            rowidx = _splat_i32(g * 16) + _iota16()
            for h in range(2):
                hg = _splat_i32(2 * c + h)
                a_s = plsc.load_gather(astab_v, [hg, tsrc16])
                a_d = plsc.load_gather(adtab_v, [hg, tdst16])
                a_e = plsc.load_gather(ae_v, [rowidx, _splat_i32(2 * c + h)])
                al = a_s + a_d + a_e
                al = jnp.maximum(al, 0.2 * al)
                ex = jnp.exp(al)
                sidx = dst16 * 32 + tsrc16 + (h * 16)
                osl = pl.ds((g * 2 + h) * 16, 16)
                val_v[osl] = ex
                idx_v[osl] = sidx
            return carry2

        lax.fori_loop(0, KP1 // 16, group, 0)
        pltpu.sync_copy(val_v, S_sp.at[idx_v], add=True)
        return carry

    lax.fori_loop(0, P1_CHUNKS, chunk, 0)
    plsc.subcore_barrier()
    pltpu.sync_copy(S_sp.at[pl.ds(s * P1_TILE_WORDS, P1_TILE_WORDS)],
                    S_out.at[c, pl.ds(s * P1_TILE_WORDS, P1_TILE_WORDS)])


def _p1_layer1_scatter(x_pad, src_pad, dst_pad, ae, astab, adtab):
    f = pl.kernel(
        _p1_body,
        out_type=jax.ShapeDtypeStruct((NC, NP * 32), jnp.float32),
        mesh=_sc_mesh(),
        compiler_params=pltpu.CompilerParams(use_tc_tiling_on_sc=False, needs_layout_passes=False),
        scratch_types=[
            pltpu.VMEM((KP1,), jnp.int32),
            pltpu.VMEM((KP1,), jnp.int32),
            pltpu.VMEM((KP1,), jnp.int32),
            pltpu.VMEM((KP1,), jnp.int32),
            pltpu.VMEM((KP1, 5), jnp.float32),
            pltpu.VMEM((HEADS, NUM_TYPES), jnp.float32),
            pltpu.VMEM((HEADS, NUM_TYPES), jnp.float32),
            pltpu.VMEM((2 * KP1,), jnp.float32),
            pltpu.VMEM((2 * KP1,), jnp.int32),
            pltpu.VMEM((P1_ZERO,), jnp.float32),
            pltpu.VMEM_SHARED((NP * 32,), jnp.float32),
            pltpu.SemaphoreType.DMA,
            pltpu.SemaphoreType.DMA,
        ],
    )
    return f(x_pad, src_pad, dst_pad, ae, astab, adtab)


P3_CHUNKS = EP // NW // KE
P3_ROWS = NP // NS


def _p3_body(src_hbm, dst_hbm, ae_hbm, xh2_hbm, as2_hbm, ad2_hbm,
             numer_out, denom_out,
             src_v, dst_v, as_v, ad_v, ae_v, rows_v, ex_v, z2_v, zf_v,
             numer_sp, denom_sp, sem1, sem2, sem3):
    c = lax.axis_index("c")
    s = lax.axis_index("s")

    def zfill2(i, carry):
        z2_v[i] = jnp.zeros((16,), jnp.float32)
        return carry

    lax.fori_loop(0, 391, zfill2, 0)

    def zfillf(i, carry):
        zf_v[pl.ds(i * 16, 16)] = jnp.zeros((16,), jnp.float32)
        return carry

    lax.fori_loop(0, 196, zfillf, 0)

    def zcopy(k, carry):
        pltpu.sync_copy(z2_v, numer_sp.at[pl.ds(s * P3_ROWS + k * 391, 391)])
        return carry

    lax.fori_loop(0, 8, zcopy, 0)
    pltpu.sync_copy(zf_v.at[pl.ds(0, P3_ROWS)], denom_sp.at[pl.ds(s * P3_ROWS, P3_ROWS)])
    plsc.subcore_barrier()

    def chunk(k, carry):
        base = (c * NS + s) * (EP // NW) + k * KE
        pltpu.sync_copy(src_hbm.at[pl.ds(base, KE)], src_v)
        pltpu.sync_copy(dst_hbm.at[pl.ds(base, KE)], dst_v)
        cp1 = pltpu.async_copy(as2_hbm.at[src_v], as_v, sem1)
        cp2 = pltpu.async_copy(ad2_hbm.at[dst_v], ad_v, sem2)
        cp3 = pltpu.async_copy(xh2_hbm.at[src_v], rows_v, sem3)
        pltpu.sync_copy(ae_hbm.at[pl.ds(base, KE)], ae_v)
        cp1.wait()
        cp2.wait()
        cp3.wait()

        def group(g, carry2):
            gsl = pl.ds(g * 16, 16)
            rowidx = _splat_i32(g * 16) + _iota16()
            a_e = plsc.load_gather(ae_v, [rowidx, _splat_i32(4)])
            al = as_v[gsl] + ad_v[gsl] + a_e
            al = jnp.maximum(al, 0.2 * al)
            ex = jnp.exp(al)
            ex_v[gsl] = ex
            for cc in range(HID):
                col = plsc.load_gather(rows_v, [rowidx, _splat_i32(cc)])
                plsc.store_scatter(rows_v, [rowidx, _splat_i32(cc)], col * ex)
            return carry2

        lax.fori_loop(0, KE // 16, group, 0)
        pltpu.sync_copy(rows_v, numer_sp.at[dst_v], add=True)
        pltpu.sync_copy(ex_v, denom_sp.at[dst_v], add=True)
        return carry

    lax.fori_loop(0, P3_CHUNKS, chunk, 0)
    plsc.subcore_barrier()
    pltpu.sync_copy(numer_sp.at[pl.ds(s * P3_ROWS, P3_ROWS)],
                    numer_out.at[c, pl.ds(s * P3_ROWS, P3_ROWS)])
    pltpu.sync_copy(denom_sp.at[pl.ds(s * P3_ROWS, P3_ROWS)],
                    denom_out.at[c, pl.ds(s * P3_ROWS, P3_ROWS)])


def _p3_layer2_scatter(src_pad, dst_pad, ae, xh2_pad, as2_pad, ad2_pad):
    f = pl.kernel(
        _p3_body,
        out_type=(jax.ShapeDtypeStruct((NC, NP, HID), jnp.float32),
                  jax.ShapeDtypeStruct((NC, NP), jnp.float32)),
        mesh=_sc_mesh(),
        compiler_params=pltpu.CompilerParams(use_tc_tiling_on_sc=False, needs_layout_passes=False),
        scratch_types=[
            pltpu.VMEM((KE,), jnp.int32),
            pltpu.VMEM((KE,), jnp.int32),
            pltpu.VMEM((KE,), jnp.float32),
            pltpu.VMEM((KE,), jnp.float32),
            pltpu.VMEM((KE, 5), jnp.float32),
            pltpu.VMEM((KE, HID), jnp.float32),
            pltpu.VMEM((KE,), jnp.float32),
            pltpu.VMEM((391, 16), jnp.float32),
            pltpu.VMEM((3136,), jnp.float32),
            pltpu.VMEM_SHARED((NP, HID), jnp.float32),
            pltpu.VMEM_SHARED((NP,), jnp.float32),
            pltpu.SemaphoreType.DMA,
            pltpu.SemaphoreType.DMA,
            pltpu.SemaphoreType.DMA,
        ],
    )
    return f(src_pad, dst_pad, ae, xh2_pad, as2_pad, ad2_pad)


def _p5_body(A_hbm, B_hbm, src_hbm, dst_hbm, b3_hbm, out_hbm,
             src_v, dst_v, rowsA_v, rowsB_v, b3_v, sem1, sem2):
    c = lax.axis_index("c")
    s = lax.axis_index("s")
    wid = c * NS + s
    pltpu.sync_copy(b3_hbm, b3_v)

    def chunk(k, carry):
        base = wid * (EP // NW) + k * KE
        pltpu.sync_copy(src_hbm.at[pl.ds(base, KE)], src_v)
        pltpu.sync_copy(dst_hbm.at[pl.ds(base, KE)], dst_v)
        cpA = pltpu.async_copy(A_hbm.at[src_v], rowsA_v, sem1)
        cpB = pltpu.async_copy(B_hbm.at[dst_v], rowsB_v, sem2)
        cpA.wait()
        cpB.wait()
        b3 = b3_v[...]

        def row(i, carry2):
            rowsA_v[i] = jnp.maximum(rowsA_v[i] + rowsB_v[i] + b3, 0.0)
            return carry2

        lax.fori_loop(0, KE, row, 0)
        pltpu.sync_copy(rowsA_v, out_hbm.at[pl.ds(base, KE)])
        return carry

    lax.fori_loop(0, EP // NW // KE, chunk, 0)


def _p5_edge_gather(A_pad, B_pad, src_pad, dst_pad, b3):
    f = pl.kernel(
        _p5_body,
        out_type=jax.ShapeDtypeStruct((EP, HID), jnp.float32),
        mesh=_sc_mesh(),
        compiler_params=pltpu.CompilerParams(use_tc_tiling_on_sc=False, needs_layout_passes=False),
        scratch_types=[
            pltpu.VMEM((KE,), jnp.int32),
            pltpu.VMEM((KE,), jnp.int32),
            pltpu.VMEM((KE, HID), jnp.float32),
            pltpu.VMEM((KE, HID), jnp.float32),
            pltpu.VMEM((HID,), jnp.float32),
            pltpu.SemaphoreType.DMA,
            pltpu.SemaphoreType.DMA,
        ],
    )
    return f(A_pad, B_pad, src_pad, dst_pad, b3)


def _ae_body(attr_ref, w_ref, out_ref):
    out_ref[...] = jnp.dot(attr_ref[...], w_ref[...],
                           preferred_element_type=jnp.float32)


def _p0_ae(edge_attr_pad, Aecat):
    return pl.pallas_call(
        _ae_body,
        grid=(EP // EBLK,),
        in_specs=[
            pl.BlockSpec((EBLK, EDGE_DIM), lambda i: (i, 0)),
            pl.BlockSpec((EDGE_DIM, 5), lambda i: (0, 0)),
        ],
        out_specs=pl.BlockSpec((EBLK, 5), lambda i: (i, 0)),
        out_shape=jax.ShapeDtypeStruct((EP, 5), jnp.float32),
    )(edge_attr_pad, Aecat)


def _mlp_body(hmid_ref, W4_ref, b4_ref, out_ref):
    out_ref[...] = hmid_ref[...] @ W4_ref[...] + b4_ref[...]


def _edge_mlp_pallas(hmid, W4, b4):
    return pl.pallas_call(
        _mlp_body,
        grid=(EP // EBLK,),
        in_specs=[
            pl.BlockSpec((EBLK, HID), lambda i: (i, 0)),
            pl.BlockSpec((HID, 3), lambda i: (0, 0)),
            pl.BlockSpec((1, 3), lambda i: (0, 0)),
        ],
        out_specs=pl.BlockSpec((EBLK, 3), lambda i: (i, 0)),
        out_shape=jax.ShapeDtypeStruct((EP, 3), jnp.float32),
    )(hmid, W4, b4.reshape(1, 3))


def kernel(x, edge_index, edge_attr, type_emb, enc_W, enc_b, W1, att_src1, att_dst1, We1, att_e1, b1, W2, att_src2, att_dst2, We2, att_e2, b2, W3, b3, W4, b4):
    src, dst = edge_index[0], edge_index[1]
    t = x[:, 0]

    h0_table = jax.nn.relu(type_emb @ enc_W + enc_b)
    xh1_table = (h0_table @ W1).reshape(NUM_TYPES, HEADS, HID)
    as1_t = (xh1_table * att_src1[None]).sum(-1)
    ad1_t = (xh1_table * att_dst1[None]).sum(-1)
    Ae1 = (We1.reshape(EDGE_DIM, HEADS, HID) * att_e1[None]).sum(-1)
    Ae2 = (We2.reshape(EDGE_DIM, 1, HID) * att_e2[None]).sum(-1)
    Aecat = jnp.concatenate([Ae1, Ae2], axis=1)
    astab = as1_t.T
    adtab = ad1_t.T

    src_pad = jnp.concatenate([src, jnp.full((EP - E,), N, src.dtype)])
    dst_pad = jnp.concatenate([dst, jnp.full((EP - E,), N, dst.dtype)])
    x_pad = jnp.concatenate([t, jnp.zeros((NP - N,), t.dtype)])
    edge_attr_pad = jnp.concatenate([edge_attr, jnp.zeros((EP - E, EDGE_DIM), jnp.float32)])

    ae = _p0_ae(edge_attr_pad, Aecat)

    S_out = _p1_layer1_scatter(x_pad, src_pad, dst_pad, ae, astab, adtab)
    S = S_out.reshape(NC, NP, 2, NUM_TYPES)[:, :N].transpose(1, 0, 2, 3).reshape(N, HEADS, NUM_TYPES)

    denom1 = S.sum(-1)
    out1 = jnp.einsum('nht,thc->nhc', S, xh1_table) / (denom1[..., None] + 1e-16)
    h1 = jax.nn.elu(out1.reshape(N, HEADS * HID) + b1)
    xh2 = h1 @ W2
    as2 = xh2 @ att_src2[0]
    ad2 = xh2 @ att_dst2[0]

    xh2_pad = jnp.concatenate([xh2, jnp.zeros((NP - N, HID), jnp.float32)])
    as2_pad = jnp.concatenate([as2, jnp.zeros((NP - N,), jnp.float32)])
    ad2_pad = jnp.concatenate([ad2, jnp.zeros((NP - N,), jnp.float32)])
    numer_p, denom_p = _p3_layer2_scatter(src_pad, dst_pad, ae, xh2_pad, as2_pad, ad2_pad)
    numer2 = numer_p[0, :N] + numer_p[1, :N]
    denom2 = denom_p[0, :N] + denom_p[1, :N]

    h2 = jax.nn.elu(numer2 / (denom2[:, None] + 1e-16) + b2)
    A = h2 @ W3[:HID]
    B = h2 @ W3[HID:]

    A_pad = jnp.concatenate([A, jnp.zeros((NP - N, HID), jnp.float32)])
    B_pad = jnp.concatenate([B, jnp.zeros((NP - N, HID), jnp.float32)])
    hmid = _p5_edge_gather(A_pad, B_pad, src_pad, dst_pad, b3)

    return _edge_mlp_pallas(hmid, W4, b4)[:E]


# speedup vs baseline: 21.6672x; 1.0899x over previous
import functools
import jax
import jax.numpy as jnp
from jax import lax
from jax.experimental import pallas as pl
from jax.experimental.pallas import tpu as pltpu
from jax.experimental.pallas import tpu_sc as plsc

N = 50000
E = 800000
NUM_TYPES = 16
EMB = 64
HID = 16
HEADS = 4
EDGE_DIM = 3

NC = 2
NS = 16
NW = NC * NS

NP = 50048
EP = 819200
KE = 1600

EBLK = 8192


def _elu(v):
    return jnp.where(v > 0, v, jnp.exp(jnp.minimum(v, 0.0)) - 1.0)


def _sc_mesh():
    return plsc.VectorSubcoreMesh(
        core_axis_name="c", subcore_axis_name="s", num_cores=NC, num_subcores=NS)


def _splat_i32(v):
    return jnp.zeros((16,), jnp.int32) + v


def _iota16():
    return lax.iota(jnp.int32, 16)


KP1 = 800
P1_CHUNKS = EP // NS // KP1
P1_TILE_WORDS = NP * 32 // NS
P1_ZERO = 6256


def _p1_body(x_hbm, src_hbm, dst_hbm, ae_hbm, astab_hbm, adtab_hbm, S_out,
             src_v, dst_v, tsrc_v, tdst_v, ae_v, astab_v, adtab_v,
             val_v, idx_v, zero_v, S_sp, sem1, sem2):
    c = lax.axis_index("c")
    s = lax.axis_index("s")

    def zfill(i, carry):
        zero_v[pl.ds(i * 16, 16)] = jnp.zeros((16,), jnp.float32)
        return carry

    lax.fori_loop(0, P1_ZERO // 16, zfill, 0)

    def zcopy(k, carry):
        pltpu.sync_copy(zero_v, S_sp.at[pl.ds(s * P1_TILE_WORDS + k * P1_ZERO, P1_ZERO)])
        return carry

    lax.fori_loop(0, NS, zcopy, 0)
    pltpu.sync_copy(astab_hbm, astab_v)
    pltpu.sync_copy(adtab_hbm, adtab_v)
    plsc.subcore_barrier()

    def chunk(k, carry):
        base = s * (EP // NS) + k * KP1
        pltpu.sync_copy(src_hbm.at[pl.ds(base, KP1)], src_v)
        pltpu.sync_copy(dst_hbm.at[pl.ds(base, KP1)], dst_v)
        cp1 = pltpu.async_copy(x_hbm.at[src_v], tsrc_v, sem1)
        cp2 = pltpu.async_copy(x_hbm.at[dst_v], tdst_v, sem2)
        pltpu.sync_copy(ae_hbm.at[pl.ds(base, KP1)], ae_v)
        cp1.wait()
        cp2.wait()

        def group(g, carry2):
            gsl = pl.ds(g * 16, 16)
            tsrc16 = tsrc_v[gsl]
            tdst16 = tdst_v[gsl]
            dst16 = dst_v[gsl]
            rowidx = _splat_i32(g * 16) + _iota16()
            for h in range(2):
                hg = _splat_i32(2 * c + h)
                a_s = plsc.load_gather(astab_v, [hg, tsrc16])
                a_d = plsc.load_gather(adtab_v, [hg, tdst16])
                a_e = plsc.load_gather(ae_v, [rowidx, _splat_i32(2 * c + h)])
                al = a_s + a_d + a_e
                al = jnp.maximum(al, 0.2 * al)
                ex = jnp.exp(al)
                sidx = dst16 * 32 + tsrc16 + (h * 16)
                osl = pl.ds((g * 2 + h) * 16, 16)
                val_v[osl] = ex
                idx_v[osl] = sidx
            return carry2

        lax.fori_loop(0, KP1 // 16, group, 0)
        pltpu.sync_copy(val_v, S_sp.at[idx_v], add=True)
        return carry

    lax.fori_loop(0, P1_CHUNKS, chunk, 0)
    plsc.subcore_barrier()
    pltpu.sync_copy(S_sp.at[pl.ds(s * P1_TILE_WORDS, P1_TILE_WORDS)],
                    S_out.at[c, pl.ds(s * P1_TILE_WORDS, P1_TILE_WORDS)])


def _p1_layer1_scatter(x_pad, src_pad, dst_pad, ae, astab, adtab):
    f = pl.kernel(
        _p1_body,
        out_type=jax.ShapeDtypeStruct((NC, NP * 32), jnp.float32),
        mesh=_sc_mesh(),
        compiler_params=pltpu.CompilerParams(use_tc_tiling_on_sc=False, needs_layout_passes=False),
        scratch_types=[
            pltpu.VMEM((KP1,), jnp.int32),
            pltpu.VMEM((KP1,), jnp.int32),
            pltpu.VMEM((KP1,), jnp.int32),
            pltpu.VMEM((KP1,), jnp.int32),
            pltpu.VMEM((KP1, 5), jnp.float32),
            pltpu.VMEM((HEADS, NUM_TYPES), jnp.float32),
            pltpu.VMEM((HEADS, NUM_TYPES), jnp.float32),
            pltpu.VMEM((2 * KP1,), jnp.float32),
            pltpu.VMEM((2 * KP1,), jnp.int32),
            pltpu.VMEM((P1_ZERO,), jnp.float32),
            pltpu.VMEM_SHARED((NP * 32,), jnp.float32),
            pltpu.SemaphoreType.DMA,
            pltpu.SemaphoreType.DMA,
        ],
    )
    return f(x_pad, src_pad, dst_pad, ae, astab, adtab)


P3_CHUNKS = EP // NW // KE
P3_ROWS = NP // NS


def _p3_body(src_hbm, dst_hbm, ae_hbm, xh2_hbm, as2_hbm, ad2_hbm,
             numer_out, denom_out,
             src_v, dst_v, as_v, ad_v, ae_v, rows_v, ex_v, z2_v, zf_v,
             numer_sp, denom_sp, sem1, sem2, sem3):
    c = lax.axis_index("c")
    s = lax.axis_index("s")

    def zfill2(i, carry):
        z2_v[i] = jnp.zeros((16,), jnp.float32)
        return carry

    lax.fori_loop(0, 391, zfill2, 0)

    def zfillf(i, carry):
        zf_v[pl.ds(i * 16, 16)] = jnp.zeros((16,), jnp.float32)
        return carry

    lax.fori_loop(0, 196, zfillf, 0)

    def zcopy(k, carry):
        pltpu.sync_copy(z2_v, numer_sp.at[pl.ds(s * P3_ROWS + k * 391, 391)])
        return carry

    lax.fori_loop(0, 8, zcopy, 0)
    pltpu.sync_copy(zf_v.at[pl.ds(0, P3_ROWS)], denom_sp.at[pl.ds(s * P3_ROWS, P3_ROWS)])
    plsc.subcore_barrier()

    def chunk(k, carry):
        base = (c * NS + s) * (EP // NW) + k * KE
        pltpu.sync_copy(src_hbm.at[pl.ds(base, KE)], src_v)
        pltpu.sync_copy(dst_hbm.at[pl.ds(base, KE)], dst_v)
        cp1 = pltpu.async_copy(as2_hbm.at[src_v], as_v, sem1)
        cp2 = pltpu.async_copy(ad2_hbm.at[dst_v], ad_v, sem2)
        cp3 = pltpu.async_copy(xh2_hbm.at[src_v], rows_v, sem3)
        pltpu.sync_copy(ae_hbm.at[pl.ds(base, KE)], ae_v)
        cp1.wait()
        cp2.wait()
        cp3.wait()

        def group(g, carry2):
            gsl = pl.ds(g * 16, 16)
            rowidx = _splat_i32(g * 16) + _iota16()
            a_e = plsc.load_gather(ae_v, [rowidx, _splat_i32(4)])
            al = as_v[gsl] + ad_v[gsl] + a_e
            al = jnp.maximum(al, 0.2 * al)
            ex = jnp.exp(al)
            ex_v[gsl] = ex
            for cc in range(HID):
                col = plsc.load_gather(rows_v, [rowidx, _splat_i32(cc)])
                plsc.store_scatter(rows_v, [rowidx, _splat_i32(cc)], col * ex)
            return carry2

        lax.fori_loop(0, KE // 16, group, 0)
        pltpu.sync_copy(rows_v, numer_sp.at[dst_v], add=True)
        pltpu.sync_copy(ex_v, denom_sp.at[dst_v], add=True)
        return carry

    lax.fori_loop(0, P3_CHUNKS, chunk, 0)
    plsc.subcore_barrier()
    pltpu.sync_copy(numer_sp.at[pl.ds(s * P3_ROWS, P3_ROWS)],
                    numer_out.at[c, pl.ds(s * P3_ROWS, P3_ROWS)])
    pltpu.sync_copy(denom_sp.at[pl.ds(s * P3_ROWS, P3_ROWS)],
                    denom_out.at[c, pl.ds(s * P3_ROWS, P3_ROWS)])


def _p3_layer2_scatter(src_pad, dst_pad, ae, xh2_pad, as2_pad, ad2_pad):
    f = pl.kernel(
        _p3_body,
        out_type=(jax.ShapeDtypeStruct((NC, NP, HID), jnp.float32),
                  jax.ShapeDtypeStruct((NC, NP), jnp.float32)),
        mesh=_sc_mesh(),
        compiler_params=pltpu.CompilerParams(use_tc_tiling_on_sc=False, needs_layout_passes=False),
        scratch_types=[
            pltpu.VMEM((KE,), jnp.int32),
            pltpu.VMEM((KE,), jnp.int32),
            pltpu.VMEM((KE,), jnp.float32),
            pltpu.VMEM((KE,), jnp.float32),
            pltpu.VMEM((KE, 5), jnp.float32),
            pltpu.VMEM((KE, HID), jnp.float32),
            pltpu.VMEM((KE,), jnp.float32),
            pltpu.VMEM((391, 16), jnp.float32),
            pltpu.VMEM((3136,), jnp.float32),
            pltpu.VMEM_SHARED((NP, HID), jnp.float32),
            pltpu.VMEM_SHARED((NP,), jnp.float32),
            pltpu.SemaphoreType.DMA,
            pltpu.SemaphoreType.DMA,
            pltpu.SemaphoreType.DMA,
        ],
    )
    return f(src_pad, dst_pad, ae, xh2_pad, as2_pad, ad2_pad)


def _p5_body(A_hbm, B_hbm, src_hbm, dst_hbm, b3_hbm, out_hbm,
             src_v, dst_v, rowsA_v, rowsB_v, b3_v, sem1, sem2):
    c = lax.axis_index("c")
    s = lax.axis_index("s")
    wid = c * NS + s
    pltpu.sync_copy(b3_hbm, b3_v)

    def chunk(k, carry):
        base = wid * (EP // NW) + k * KE
        pltpu.sync_copy(src_hbm.at[pl.ds(base, KE)], src_v)
        pltpu.sync_copy(dst_hbm.at[pl.ds(base, KE)], dst_v)
        cpA = pltpu.async_copy(A_hbm.at[src_v], rowsA_v, sem1)
        cpB = pltpu.async_copy(B_hbm.at[dst_v], rowsB_v, sem2)
        cpA.wait()
        cpB.wait()
        b3 = b3_v[...]

        def row(i, carry2):
            rowsA_v[i] = jnp.maximum(rowsA_v[i] + rowsB_v[i] + b3, 0.0)
            return carry2

        lax.fori_loop(0, KE, row, 0)
        pltpu.sync_copy(rowsA_v, out_hbm.at[pl.ds(base, KE)])
        return carry

    lax.fori_loop(0, EP // NW // KE, chunk, 0)


def _p5_edge_gather(A_pad, B_pad, src_pad, dst_pad, b3):
    f = pl.kernel(
        _p5_body,
        out_type=jax.ShapeDtypeStruct((EP, HID), jnp.float32),
        mesh=_sc_mesh(),
        compiler_params=pltpu.CompilerParams(use_tc_tiling_on_sc=False, needs_layout_passes=False),
        scratch_types=[
            pltpu.VMEM((KE,), jnp.int32),
            pltpu.VMEM((KE,), jnp.int32),
            pltpu.VMEM((KE, HID), jnp.float32),
            pltpu.VMEM((KE, HID), jnp.float32),
            pltpu.VMEM((HID,), jnp.float32),
            pltpu.SemaphoreType.DMA,
            pltpu.SemaphoreType.DMA,
        ],
    )
    return f(A_pad, B_pad, src_pad, dst_pad, b3)


def _ae_body(attr_ref, w_ref, out_ref):
    out_ref[...] = jnp.dot(attr_ref[...], w_ref[...],
                           preferred_element_type=jnp.float32)


def _p0_ae(edge_attr_pad, Aecat):
    return pl.pallas_call(
        _ae_body,
        grid=(EP // EBLK,),
        in_specs=[
            pl.BlockSpec((EBLK, EDGE_DIM), lambda i: (i, 0)),
            pl.BlockSpec((EDGE_DIM, 5), lambda i: (0, 0)),
        ],
        out_specs=pl.BlockSpec((EBLK, 5), lambda i: (i, 0)),
        out_shape=jax.ShapeDtypeStruct((EP, 5), jnp.float32),
    )(edge_attr_pad, Aecat)


def _mlp_body(hmid_ref, W4_ref, b4_ref, out_ref):
    out_ref[...] = hmid_ref[...] @ W4_ref[...] + b4_ref[...]


OBLK = 8000


def _edge_mlp_pallas(hmid, W4, b4):
    return pl.pallas_call(
        _mlp_body,
        grid=(E // OBLK,),
        in_specs=[
            pl.BlockSpec((OBLK, HID), lambda i: (i, 0)),
            pl.BlockSpec((HID, 3), lambda i: (0, 0)),
            pl.BlockSpec((1, 3), lambda i: (0, 0)),
        ],
        out_specs=pl.BlockSpec((OBLK, 3), lambda i: (i, 0)),
        out_shape=jax.ShapeDtypeStruct((E, 3), jnp.float32),
    )(hmid, W4, b4.reshape(1, 3))


NBLK = 3128


def _p2_body(S0_ref, S1_ref, W01_ref, W23_ref, R01_ref, R23_ref, b1_ref,
             W2_ref, asv_ref, adv_ref, xh2_ref, as2_ref, ad2_ref):
    S0 = S0_ref[...]
    S1 = S1_ref[...]
    dot = functools.partial(jnp.dot, preferred_element_type=jnp.float32)
    numer = dot(S0, W01_ref[...]) + dot(S1, W23_ref[...])
    denomr = dot(S0, R01_ref[...]) + dot(S1, R23_ref[...])
    h1 = _elu(numer / (denomr + 1e-16) + b1_ref[...])
    xh2 = dot(h1, W2_ref[...])
    xh2_ref[...] = xh2
    as2_ref[...] = dot(xh2, asv_ref[...])
    ad2_ref[...] = dot(xh2, adv_ref[...])


def _p2_combine(S0, S1, W01, W23, R01, R23, b1, W2, asv, adv):
    mat = lambda r, c: pl.BlockSpec((r, c), lambda i: (0, 0))
    return pl.pallas_call(
        _p2_body,
        grid=(NP // NBLK,),
        in_specs=[
            pl.BlockSpec((NBLK, 32), lambda i: (i, 0)),
            pl.BlockSpec((NBLK, 32), lambda i: (i, 0)),
            mat(32, 64), mat(32, 64), mat(32, 64), mat(32, 64),
            mat(1, 64), mat(64, HID), mat(HID, 1), mat(HID, 1),
        ],
        out_specs=[
            pl.BlockSpec((NBLK, HID), lambda i: (i, 0)),
            pl.BlockSpec((NBLK, 1), lambda i: (i, 0)),
            pl.BlockSpec((NBLK, 1), lambda i: (i, 0)),
        ],
        out_shape=[
            jax.ShapeDtypeStruct((NP, HID), jnp.float32),
            jax.ShapeDtypeStruct((NP, 1), jnp.float32),
            jax.ShapeDtypeStruct((NP, 1), jnp.float32),
        ],
    )(S0, S1, W01, W23, R01, R23, b1, W2, asv, adv)


def _p4_body(n0_ref, n1_ref, d0_ref, d1_ref, b2_ref, W3t_ref, W3b_ref,
             A_ref, B_ref):
    dot = functools.partial(jnp.dot, preferred_element_type=jnp.float32)
    numer = n0_ref[...] + n1_ref[...]
    denom = d0_ref[...] + d1_ref[...]
    h2 = _elu(numer / (denom + 1e-16) + b2_ref[...])
    A_ref[...] = dot(h2, W3t_ref[...])
    B_ref[...] = dot(h2, W3b_ref[...])


def _p4_combine(n0, n1, d0, d1, b2, W3t, W3b):
    mat = lambda r, c: pl.BlockSpec((r, c), lambda i: (0, 0))
    return pl.pallas_call(
        _p4_body,
        grid=(NP // NBLK,),
        in_specs=[
            pl.BlockSpec((NBLK, HID), lambda i: (i, 0)),
            pl.BlockSpec((NBLK, HID), lambda i: (i, 0)),
            pl.BlockSpec((NBLK, 1), lambda i: (i, 0)),
            pl.BlockSpec((NBLK, 1), lambda i: (i, 0)),
            mat(1, HID), mat(HID, HID), mat(HID, HID),
        ],
        out_specs=[
            pl.BlockSpec((NBLK, HID), lambda i: (i, 0)),
            pl.BlockSpec((NBLK, HID), lambda i: (i, 0)),
        ],
        out_shape=[
            jax.ShapeDtypeStruct((NP, HID), jnp.float32),
            jax.ShapeDtypeStruct((NP, HID), jnp.float32),
        ],
    )(n0, n1, d0, d1, b2, W3t, W3b)


def kernel(x, edge_index, edge_attr, type_emb, enc_W, enc_b, W1, att_src1, att_dst1, We1, att_e1, b1, W2, att_src2, att_dst2, We2, att_e2, b2, W3, b3, W4, b4):
    src, dst = edge_index[0], edge_index[1]
    t = x[:, 0]

    h0_table = jax.nn.relu(type_emb @ enc_W + enc_b)
    xh1_table = (h0_table @ W1).reshape(NUM_TYPES, HEADS, HID)
    as1_t = (xh1_table * att_src1[None]).sum(-1)
    ad1_t = (xh1_table * att_dst1[None]).sum(-1)
    Ae1 = (We1.reshape(EDGE_DIM, HEADS, HID) * att_e1[None]).sum(-1)
    Ae2 = (We2.reshape(EDGE_DIM, 1, HID) * att_e2[None]).sum(-1)
    Aecat = jnp.concatenate([Ae1, Ae2], axis=1)
    astab = as1_t.T
    adtab = ad1_t.T

    src_pad = jnp.concatenate([src, jnp.full((EP - E,), N, src.dtype)])
    dst_pad = jnp.concatenate([dst, jnp.full((EP - E,), N, dst.dtype)])
    x_pad = jnp.concatenate([t, jnp.zeros((NP - N,), t.dtype)])
    edge_attr_pad = jnp.concatenate([edge_attr, jnp.zeros((EP - E, EDGE_DIM), jnp.float32)])

    ae = _p0_ae(edge_attr_pad, Aecat)

    S_out = _p1_layer1_scatter(x_pad, src_pad, dst_pad, ae, astab, adtab)
    S0 = S_out[0].reshape(NP, 32)
    S1 = S_out[1].reshape(NP, 32)

    eye4 = jnp.eye(HEADS, dtype=jnp.float32)
    BigW = jnp.einsum('thc,hg->htgc', xh1_table, eye4).reshape(64, 64)
    R = jnp.broadcast_to(eye4[:, None, :, None], (HEADS, NUM_TYPES, HEADS, HID)).reshape(64, 64)
    xh2_pad, as2_p, ad2_p = _p2_combine(
        S0, S1, BigW[:32], BigW[32:], R[:32], R[32:],
        b1.reshape(1, 64), W2, att_src2[0].reshape(HID, 1), att_dst2[0].reshape(HID, 1))

    numer_p, denom_p = _p3_layer2_scatter(
        src_pad, dst_pad, ae, xh2_pad, as2_p.reshape(NP), ad2_p.reshape(NP))

    A_pad, B_pad = _p4_combine(
        numer_p[0], numer_p[1], denom_p[0].reshape(NP, 1), denom_p[1].reshape(NP, 1),
        b2.reshape(1, HID), W3[:HID], W3[HID:])

    hmid = _p5_edge_gather(A_pad, B_pad, src_pad, dst_pad, b3)

    return _edge_mlp_pallas(hmid, W4, b4)


# speedup vs baseline: 48.0684x; 2.2185x over previous
import functools
import jax
import jax.numpy as jnp
from jax import lax
from jax.experimental import pallas as pl
from jax.experimental.pallas import tpu as pltpu
from jax.experimental.pallas import tpu_sc as plsc

N = 50000
E = 800000
NUM_TYPES = 16
EMB = 64
HID = 16
HEADS = 4
EDGE_DIM = 3

NC = 2
NS = 16
NW = NC * NS

NP = 50048
EP = 819200
KE = 1600

EBLK = 8192


def _elu(v):
    return jnp.where(v > 0, v, jnp.exp(jnp.minimum(v, 0.0)) - 1.0)


def _sc_mesh():
    return plsc.VectorSubcoreMesh(
        core_axis_name="c", subcore_axis_name="s", num_cores=NC, num_subcores=NS)


def _splat_i32(v):
    return jnp.zeros((16,), jnp.int32) + v


def _iota16():
    return lax.iota(jnp.int32, 16)


KP1 = 800
P1_CHUNKS = EP // NS // KP1
P1_TILE_WORDS = NP * 32 // NS
P1_ZERO = 6256


def _p1_body(x_hbm, src_hbm, dst_hbm, attrT_hbm, aetab_hbm, astab_hbm, adtab_hbm, S_out,
             src_v, dst_v, tsrc_v, tdst_v, a0_v, a1_v, a2_v, aetab_v, astab_v, adtab_v,
             val_v, idx_v, zero_v, S_sp, sem1, sem2):
    c = lax.axis_index("c")
    s = lax.axis_index("s")

    def zfill(i, carry):
        zero_v[pl.ds(i * 16, 16)] = jnp.zeros((16,), jnp.float32)
        return carry

    lax.fori_loop(0, P1_ZERO // 16, zfill, 0)

    def zcopy(k, carry):
        pltpu.sync_copy(zero_v, S_sp.at[pl.ds(s * P1_TILE_WORDS + k * P1_ZERO, P1_ZERO)])
        return carry

    lax.fori_loop(0, NS, zcopy, 0)
    pltpu.sync_copy(astab_hbm, astab_v)
    pltpu.sync_copy(adtab_hbm, adtab_v)
    pltpu.sync_copy(aetab_hbm, aetab_v)
    sae = [[plsc.load_gather(aetab_v, [_splat_i32(d * 5 + 2 * c + h)])
            for h in range(2)] for d in range(3)]
    plsc.subcore_barrier()

    def chunk(k, carry):
        base = s * (EP // NS) + k * KP1
        pltpu.sync_copy(src_hbm.at[pl.ds(base, KP1)], src_v)
        pltpu.sync_copy(dst_hbm.at[pl.ds(base, KP1)], dst_v)
        cp1 = pltpu.async_copy(x_hbm.at[src_v], tsrc_v, sem1)
        cp2 = pltpu.async_copy(x_hbm.at[dst_v], tdst_v, sem2)
        pltpu.sync_copy(attrT_hbm.at[0, pl.ds(base, KP1)], a0_v)
        pltpu.sync_copy(attrT_hbm.at[1, pl.ds(base, KP1)], a1_v)
        pltpu.sync_copy(attrT_hbm.at[2, pl.ds(base, KP1)], a2_v)
        cp1.wait()
        cp2.wait()

        def group(g, carry2):
            gsl = pl.ds(g * 16, 16)
            tsrc16 = tsrc_v[gsl]
            tdst16 = tdst_v[gsl]
            dst16 = dst_v[gsl]
            a0 = a0_v[gsl]
            a1 = a1_v[gsl]
            a2 = a2_v[gsl]
            for h in range(2):
                hg = _splat_i32(2 * c + h)
                a_s = plsc.load_gather(astab_v, [hg, tsrc16])
                a_d = plsc.load_gather(adtab_v, [hg, tdst16])
                a_e = a0 * sae[0][h] + a1 * sae[1][h] + a2 * sae[2][h]
                al = a_s + a_d + a_e
                al = jnp.maximum(al, 0.2 * al)
                ex = jnp.exp(al)
                sidx = dst16 * 32 + tsrc16 + (h * 16)
                osl = pl.ds((g * 2 + h) * 16, 16)
                val_v[osl] = ex
                idx_v[osl] = sidx
            return carry2

        lax.fori_loop(0, KP1 // 16, group, 0)
        pltpu.sync_copy(val_v, S_sp.at[idx_v], add=True)
        return carry

    lax.fori_loop(0, P1_CHUNKS, chunk, 0)
    plsc.subcore_barrier()
    pltpu.sync_copy(S_sp.at[pl.ds(s * P1_TILE_WORDS, P1_TILE_WORDS)],
                    S_out.at[c, pl.ds(s * P1_TILE_WORDS, P1_TILE_WORDS)])


def _p1_layer1_scatter(x_pad, src_pad, dst_pad, attrT_pad, aetab, astab, adtab):
    f = pl.kernel(
        _p1_body,
        out_type=jax.ShapeDtypeStruct((NC, NP * 32), jnp.float32),
        mesh=_sc_mesh(),
        compiler_params=pltpu.CompilerParams(use_tc_tiling_on_sc=False, needs_layout_passes=False),
        scratch_types=[
            pltpu.VMEM((KP1,), jnp.int32),
            pltpu.VMEM((KP1,), jnp.int32),
            pltpu.VMEM((KP1,), jnp.int32),
            pltpu.VMEM((KP1,), jnp.int32),
            pltpu.VMEM((KP1,), jnp.float32),
            pltpu.VMEM((KP1,), jnp.float32),
            pltpu.VMEM((KP1,), jnp.float32),
            pltpu.VMEM((16,), jnp.float32),
            pltpu.VMEM((HEADS, NUM_TYPES), jnp.float32),
            pltpu.VMEM((HEADS, NUM_TYPES), jnp.float32),
            pltpu.VMEM((2 * KP1,), jnp.float32),
            pltpu.VMEM((2 * KP1,), jnp.int32),
            pltpu.VMEM((P1_ZERO,), jnp.float32),
            pltpu.VMEM_SHARED((NP * 32,), jnp.float32),
            pltpu.SemaphoreType.DMA,
            pltpu.SemaphoreType.DMA,
        ],
    )
    return f(x_pad, src_pad, dst_pad, attrT_pad, aetab, astab, adtab)


P3_CHUNKS = EP // NW // KE
P3_ROWS = NP // NS


def _p3_body(src_hbm, dst_hbm, attrT_hbm, aetab_hbm, xh2_hbm, as2_hbm, ad2_hbm,
             numer_out, denom_out,
             src_v, dst_v, as_v, ad_v, a0_v, a1_v, a2_v, aetab_v, rows_v, ex_v, z2_v, zf_v,
             numer_sp, denom_sp, sem1, sem2, sem3):
    c = lax.axis_index("c")
    s = lax.axis_index("s")

    def zfill2(i, carry):
        z2_v[i] = jnp.zeros((16,), jnp.float32)
        return carry

    lax.fori_loop(0, 391, zfill2, 0)

    def zfillf(i, carry):
        zf_v[pl.ds(i * 16, 16)] = jnp.zeros((16,), jnp.float32)
        return carry

    lax.fori_loop(0, 196, zfillf, 0)

    def zcopy(k, carry):
        pltpu.sync_copy(z2_v, numer_sp.at[pl.ds(s * P3_ROWS + k * 391, 391)])
        return carry

    lax.fori_loop(0, 8, zcopy, 0)
    pltpu.sync_copy(zf_v.at[pl.ds(0, P3_ROWS)], denom_sp.at[pl.ds(s * P3_ROWS, P3_ROWS)])
    pltpu.sync_copy(aetab_hbm, aetab_v)
    sae = [plsc.load_gather(aetab_v, [_splat_i32(d * 5 + 4)]) for d in range(3)]
    plsc.subcore_barrier()

    def chunk(k, carry):
        base = (c * NS + s) * (EP // NW) + k * KE
        pltpu.sync_copy(src_hbm.at[pl.ds(base, KE)], src_v)
        pltpu.sync_copy(dst_hbm.at[pl.ds(base, KE)], dst_v)
        cp1 = pltpu.async_copy(as2_hbm.at[src_v], as_v, sem1)
        cp2 = pltpu.async_copy(ad2_hbm.at[dst_v], ad_v, sem2)
        cp3 = pltpu.async_copy(xh2_hbm.at[src_v], rows_v, sem3)
        pltpu.sync_copy(attrT_hbm.at[0, pl.ds(base, KE)], a0_v)
        pltpu.sync_copy(attrT_hbm.at[1, pl.ds(base, KE)], a1_v)
        pltpu.sync_copy(attrT_hbm.at[2, pl.ds(base, KE)], a2_v)
        cp1.wait()
        cp2.wait()
        cp3.wait()

        def group(g, carry2):
            gsl = pl.ds(g * 16, 16)
            rowidx = _splat_i32(g * 16) + _iota16()
            a_e = a0_v[gsl] * sae[0] + a1_v[gsl] * sae[1] + a2_v[gsl] * sae[2]
            al = as_v[gsl] + ad_v[gsl] + a_e
            al = jnp.maximum(al, 0.2 * al)
            ex = jnp.exp(al)
            ex_v[gsl] = ex
            for cc in range(HID):
                col = plsc.load_gather(rows_v, [rowidx, _splat_i32(cc)])
                plsc.store_scatter(rows_v, [rowidx, _splat_i32(cc)], col * ex)
            return carry2

        lax.fori_loop(0, KE // 16, group, 0)
        pltpu.sync_copy(rows_v, numer_sp.at[dst_v], add=True)
        pltpu.sync_copy(ex_v, denom_sp.at[dst_v], add=True)
        return carry

    lax.fori_loop(0, P3_CHUNKS, chunk, 0)
    plsc.subcore_barrier()
    pltpu.sync_copy(numer_sp.at[pl.ds(s * P3_ROWS, P3_ROWS)],
                    numer_out.at[c, pl.ds(s * P3_ROWS, P3_ROWS)])
    pltpu.sync_copy(denom_sp.at[pl.ds(s * P3_ROWS, P3_ROWS)],
                    denom_out.at[c, pl.ds(s * P3_ROWS, P3_ROWS)])


def _p3_layer2_scatter(src_pad, dst_pad, attrT_pad, aetab, xh2_pad, as2_pad, ad2_pad):
    f = pl.kernel(
        _p3_body,
        out_type=(jax.ShapeDtypeStruct((NC, NP, HID), jnp.float32),
                  jax.ShapeDtypeStruct((NC, NP), jnp.float32)),
        mesh=_sc_mesh(),
        compiler_params=pltpu.CompilerParams(use_tc_tiling_on_sc=False, needs_layout_passes=False),
        scratch_types=[
            pltpu.VMEM((KE,), jnp.int32),
            pltpu.VMEM((KE,), jnp.int32),
            pltpu.VMEM((KE,), jnp.float32),
            pltpu.VMEM((KE,), jnp.float32),
            pltpu.VMEM((KE,), jnp.float32),
            pltpu.VMEM((KE,), jnp.float32),
            pltpu.VMEM((KE,), jnp.float32),
            pltpu.VMEM((16,), jnp.float32),
            pltpu.VMEM((KE, HID), jnp.float32),
            pltpu.VMEM((KE,), jnp.float32),
            pltpu.VMEM((391, 16), jnp.float32),
            pltpu.VMEM((3136,), jnp.float32),
            pltpu.VMEM_SHARED((NP, HID), jnp.float32),
            pltpu.VMEM_SHARED((NP,), jnp.float32),
            pltpu.SemaphoreType.DMA,
            pltpu.SemaphoreType.DMA,
            pltpu.SemaphoreType.DMA,
        ],
    )
    return f(src_pad, dst_pad, attrT_pad, aetab, xh2_pad, as2_pad, ad2_pad)


def _p5_body(A_hbm, B_hbm, src_hbm, dst_hbm, b3_hbm, out_hbm,
             src_v, dst_v, rowsA_v, rowsB_v, b3_v, sem1, sem2):
    c = lax.axis_index("c")
    s = lax.axis_index("s")
    wid = c * NS + s
    pltpu.sync_copy(b3_hbm, b3_v)

    def chunk(k, carry):
        base = wid * (EP // NW) + k * KE
        pltpu.sync_copy(src_hbm.at[pl.ds(base, KE)], src_v)
        pltpu.sync_copy(dst_hbm.at[pl.ds(base, KE)], dst_v)
        cpA = pltpu.async_copy(A_hbm.at[src_v], rowsA_v, sem1)
        cpB = pltpu.async_copy(B_hbm.at[dst_v], rowsB_v, sem2)
        cpA.wait()
        cpB.wait()
        b3 = b3_v[...]

        def row(i, carry2):
            rowsA_v[i] = jnp.maximum(rowsA_v[i] + rowsB_v[i] + b3, 0.0)
            return carry2

        lax.fori_loop(0, KE, row, 0)
        pltpu.sync_copy(rowsA_v, out_hbm.at[pl.ds(base, KE)])
        return carry

    lax.fori_loop(0, EP // NW // KE, chunk, 0)


def _p5_edge_gather(A_pad, B_pad, src_pad, dst_pad, b3):
    f = pl.kernel(
        _p5_body,
        out_type=jax.ShapeDtypeStruct((EP, HID), jnp.float32),
        mesh=_sc_mesh(),
        compiler_params=pltpu.CompilerParams(use_tc_tiling_on_sc=False, needs_layout_passes=False),
        scratch_types=[
            pltpu.VMEM((KE,), jnp.int32),
            pltpu.VMEM((KE,), jnp.int32),
            pltpu.VMEM((KE, HID), jnp.float32),
            pltpu.VMEM((KE, HID), jnp.float32),
            pltpu.VMEM((HID,), jnp.float32),
            pltpu.SemaphoreType.DMA,
            pltpu.SemaphoreType.DMA,
        ],
    )
    return f(A_pad, B_pad, src_pad, dst_pad, b3)


def _ae_body(attr_ref, w_ref, out_ref):
    out_ref[...] = jnp.dot(attr_ref[...], w_ref[...],
                           preferred_element_type=jnp.float32)


def _p0_ae(edge_attr_pad, Aecat):
    return pl.pallas_call(
        _ae_body,
        grid=(EP // EBLK,),
        in_specs=[
            pl.BlockSpec((EBLK, EDGE_DIM), lambda i: (i, 0)),
            pl.BlockSpec((EDGE_DIM, 5), lambda i: (0, 0)),
        ],
        out_specs=pl.BlockSpec((EBLK, 5), lambda i: (i, 0)),
        out_shape=jax.ShapeDtypeStruct((EP, 5), jnp.float32),
    )(edge_attr_pad, Aecat)


def _mlp_body(hmid_ref, W4_ref, b4_ref, out_ref):
    out_ref[...] = hmid_ref[...] @ W4_ref[...] + b4_ref[...]


OBLK = 8000


def _edge_mlp_pallas(hmid, W4, b4):
    return pl.pallas_call(
        _mlp_body,
        grid=(E // OBLK,),
        in_specs=[
            pl.BlockSpec((OBLK, HID), lambda i: (i, 0)),
            pl.BlockSpec((HID, 3), lambda i: (0, 0)),
            pl.BlockSpec((1, 3), lambda i: (0, 0)),
        ],
        out_specs=pl.BlockSpec((OBLK, 3), lambda i: (i, 0)),
        out_shape=jax.ShapeDtypeStruct((E, 3), jnp.float32),
    )(hmid, W4, b4.reshape(1, 3))


NBLK = 3128


def _p2_body(S0_ref, S1_ref, W01_ref, W23_ref, R01_ref, R23_ref, b1_ref,
             W2_ref, asv_ref, adv_ref, xh2_ref, as2_ref, ad2_ref):
    S0 = S0_ref[...]
    S1 = S1_ref[...]
    dot = functools.partial(jnp.dot, preferred_element_type=jnp.float32)
    numer = dot(S0, W01_ref[...]) + dot(S1, W23_ref[...])
    denomr = dot(S0, R01_ref[...]) + dot(S1, R23_ref[...])
    h1 = _elu(numer / (denomr + 1e-16) + b1_ref[...])
    xh2 = dot(h1, W2_ref[...])
    xh2_ref[...] = xh2
    as2_ref[...] = dot(xh2, asv_ref[...])
    ad2_ref[...] = dot(xh2, adv_ref[...])


def _p2_combine(S0, S1, W01, W23, R01, R23, b1, W2, asv, adv):
    mat = lambda r, c: pl.BlockSpec((r, c), lambda i: (0, 0))
    return pl.pallas_call(
        _p2_body,
        grid=(NP // NBLK,),
        in_specs=[
            pl.BlockSpec((NBLK, 32), lambda i: (i, 0)),
            pl.BlockSpec((NBLK, 32), lambda i: (i, 0)),
            mat(32, 64), mat(32, 64), mat(32, 64), mat(32, 64),
            mat(1, 64), mat(64, HID), mat(HID, 1), mat(HID, 1),
        ],
        out_specs=[
            pl.BlockSpec((NBLK, HID), lambda i: (i, 0)),
            pl.BlockSpec((NBLK, 1), lambda i: (i, 0)),
            pl.BlockSpec((NBLK, 1), lambda i: (i, 0)),
        ],
        out_shape=[
            jax.ShapeDtypeStruct((NP, HID), jnp.float32),
            jax.ShapeDtypeStruct((NP, 1), jnp.float32),
            jax.ShapeDtypeStruct((NP, 1), jnp.float32),
        ],
    )(S0, S1, W01, W23, R01, R23, b1, W2, asv, adv)


def _p4_body(n0_ref, n1_ref, d0_ref, d1_ref, b2_ref, W3t_ref, W3b_ref,
             A_ref, B_ref):
    dot = functools.partial(jnp.dot, preferred_element_type=jnp.float32)
    numer = n0_ref[...] + n1_ref[...]
    denom = d0_ref[...] + d1_ref[...]
    h2 = _elu(numer / (denom + 1e-16) + b2_ref[...])
    A_ref[...] = dot(h2, W3t_ref[...])
    B_ref[...] = dot(h2, W3b_ref[...])


def _p4_combine(n0, n1, d0, d1, b2, W3t, W3b):
    mat = lambda r, c: pl.BlockSpec((r, c), lambda i: (0, 0))
    return pl.pallas_call(
        _p4_body,
        grid=(NP // NBLK,),
        in_specs=[
            pl.BlockSpec((NBLK, HID), lambda i: (i, 0)),
            pl.BlockSpec((NBLK, HID), lambda i: (i, 0)),
            pl.BlockSpec((NBLK, 1), lambda i: (i, 0)),
            pl.BlockSpec((NBLK, 1), lambda i: (i, 0)),
            mat(1, HID), mat(HID, HID), mat(HID, HID),
        ],
        out_specs=[
            pl.BlockSpec((NBLK, HID), lambda i: (i, 0)),
            pl.BlockSpec((NBLK, HID), lambda i: (i, 0)),
        ],
        out_shape=[
            jax.ShapeDtypeStruct((NP, HID), jnp.float32),
            jax.ShapeDtypeStruct((NP, HID), jnp.float32),
        ],
    )(n0, n1, d0, d1, b2, W3t, W3b)


def kernel(x, edge_index, edge_attr, type_emb, enc_W, enc_b, W1, att_src1, att_dst1, We1, att_e1, b1, W2, att_src2, att_dst2, We2, att_e2, b2, W3, b3, W4, b4):
    src, dst = edge_index[0], edge_index[1]
    t = x[:, 0]

    h0_table = jax.nn.relu(type_emb @ enc_W + enc_b)
    xh1_table = (h0_table @ W1).reshape(NUM_TYPES, HEADS, HID)
    as1_t = (xh1_table * att_src1[None]).sum(-1)
    ad1_t = (xh1_table * att_dst1[None]).sum(-1)
    Ae1 = (We1.reshape(EDGE_DIM, HEADS, HID) * att_e1[None]).sum(-1)
    Ae2 = (We2.reshape(EDGE_DIM, 1, HID) * att_e2[None]).sum(-1)
    Aecat = jnp.concatenate([Ae1, Ae2], axis=1)
    astab = as1_t.T
    adtab = ad1_t.T

    src_pad = jnp.concatenate([src, jnp.full((EP - E,), N, src.dtype)])
    dst_pad = jnp.concatenate([dst, jnp.full((EP - E,), N, dst.dtype)])
    x_pad = jnp.concatenate([t, jnp.zeros((NP - N,), t.dtype)])
    attrT_pad = jnp.concatenate(
        [edge_attr.T, jnp.zeros((EDGE_DIM, EP - E), jnp.float32)], axis=1)
    aetab = jnp.concatenate([Aecat.reshape(15), jnp.zeros((1,), jnp.float32)])

    S_out = _p1_layer1_scatter(x_pad, src_pad, dst_pad, attrT_pad, aetab, astab, adtab)
    S0 = S_out[0].reshape(NP, 32)
    S1 = S_out[1].reshape(NP, 32)

    eye4 = jnp.eye(HEADS, dtype=jnp.float32)
    BigW = jnp.einsum('thc,hg->htgc', xh1_table, eye4).reshape(64, 64)
    R = jnp.broadcast_to(eye4[:, None, :, None], (HEADS, NUM_TYPES, HEADS, HID)).reshape(64, 64)
    xh2_pad, as2_p, ad2_p = _p2_combine(
        S0, S1, BigW[:32], BigW[32:], R[:32], R[32:],
        b1.reshape(1, 64), W2, att_src2[0].reshape(HID, 1), att_dst2[0].reshape(HID, 1))

    numer_p, denom_p = _p3_layer2_scatter(
        src_pad, dst_pad, attrT_pad, aetab, xh2_pad, as2_p.reshape(NP), ad2_p.reshape(NP))

    A_pad, B_pad = _p4_combine(
        numer_p[0], numer_p[1], denom_p[0].reshape(NP, 1), denom_p[1].reshape(NP, 1),
        b2.reshape(1, HID), W3[:HID], W3[HID:])

    hmid = _p5_edge_gather(A_pad, B_pad, src_pad, dst_pad, b3)

    return _edge_mlp_pallas(hmid, W4, b4)
